# baseline (speedup 1.0000x reference)
import functools

import jax
import jax.numpy as jnp
from jax.experimental import pallas as pl
from jax.experimental.pallas import tpu as pltpu
from jax.experimental.pallas import tpu_sc as plsc

MM_HIDDEN = 128
TEXT_HIDDEN = 2048
EPS = 1e-06

GATHER_WINDOW = 256
ROW_BLOCK = 1024
NUM_OUT_BUFS = 4


def _sc_gather(table, ids_flat):
    n = ids_flat.shape[0]
    ids2d = ids_flat.reshape(1, n)
    mesh = plsc.VectorSubcoreMesh(core_axis_name="core", subcore_axis_name="subcore")

    @pl.kernel(
        out_type=jax.ShapeDtypeStruct((n, MM_HIDDEN), table.dtype),
        mesh=mesh,
    )
    def gather_kernel(table_hbm, ids_hbm, out_hbm):
        def body(i_vmem, o_vmem):
            pltpu.sync_copy(table_hbm.at[i_vmem.at[0]], o_vmem)

        pltpu.emit_pipeline(
            body,
            grid=(n // GATHER_WINDOW,),
            in_specs=[pl.BlockSpec((1, GATHER_WINDOW), lambda i: (0, i))],
            out_specs=[pl.BlockSpec((GATHER_WINDOW, MM_HIDDEN), lambda i: (i, 0))],
            core_axis_name=("core", "subcore"),
            dimension_semantics=(pltpu.PARALLEL,),
        )(ids_hbm, out_hbm)

    return gather_kernel(table, ids2d)


def _tc_body(x_ref, s_ref, w_ref, o_hbm, w16_ref, g16_ref, bufs, sems,
             *, row_offset):
    @pl.when(pl.program_id(0) == 0)
    def _():
        w16 = w_ref[...].astype(jnp.bfloat16)
        w16_ref[...] = w16
        g = jax.lax.dot_general(
            w16, w16, (((1,), (1,)), ((), ())),
            preferred_element_type=jnp.float32,
        )
        g16_ref[...] = g.astype(jnp.bfloat16)

    x = x_ref[...]
    inv1 = jax.lax.rsqrt(jnp.mean(x * x, axis=1, keepdims=True) + EPS)
    y32 = x * inv1 * s_ref[...]
    y = y32.astype(jnp.bfloat16)
    t = jax.lax.dot_general(
        y, g16_ref[...], (((1,), (0,)), ((), ())),
        preferred_element_type=jnp.float32,
    )
    q = jnp.sum(t * y32, axis=1, keepdims=True)
    inv2 = jax.lax.rsqrt(q / TEXT_HIDDEN + EPS)
    y2 = (y32 * inv2).astype(jnp.bfloat16)
    z = jax.lax.dot_general(
        y2, w16_ref[...], (((1,), (0,)), ((), ())),
        preferred_element_type=jnp.float32,
    )

    i = pl.program_id(0)
    num_steps = pl.num_programs(0)
    slot = jax.lax.rem(i, NUM_OUT_BUFS)

    def _copy(s, step):
        return pltpu.make_async_copy(
            bufs.at[s],
            o_hbm.at[pl.ds(row_offset + step * ROW_BLOCK, ROW_BLOCK), :],
            sems.at[s],
        )

    @pl.when(i >= NUM_OUT_BUFS)
    def _():
        _copy(slot, i - NUM_OUT_BUFS).wait()

    bufs[slot] = z
    _copy(slot, i).start()

    @pl.when(i == num_steps - 1)
    def _():
        @pl.loop(0, NUM_OUT_BUFS)
        def _(j):
            step = num_steps - NUM_OUT_BUFS + j
            _copy(jax.lax.rem(step, NUM_OUT_BUFS), step).wait()


def _tc_norm_proj_norm(gathered, scale, weight, n_total, row_offset, out_prev):
    n = gathered.shape[0]
    body = functools.partial(_tc_body, row_offset=row_offset)
    in_specs = [
        pl.BlockSpec((ROW_BLOCK, MM_HIDDEN), lambda i: (i, 0)),
        pl.BlockSpec((1, MM_HIDDEN), lambda i: (0, 0)),
        pl.BlockSpec((MM_HIDDEN, TEXT_HIDDEN), lambda i: (0, 0)),
    ]
    args = [gathered, scale.reshape(1, MM_HIDDEN), weight]
    aliases = {}
    if out_prev is not None:
        in_specs.append(pl.BlockSpec(memory_space=pl.ANY))
        args.append(out_prev)
        aliases = {3: 0}

    def wrapped(*refs):
        if out_prev is not None:
            x_ref, s_ref, w_ref, _oin, o_hbm, *scratch = refs
        else:
            x_ref, s_ref, w_ref, o_hbm, *scratch = refs
        body(x_ref, s_ref, w_ref, o_hbm, *scratch)

    return pl.pallas_call(
        wrapped,
        grid=(n // ROW_BLOCK,),
        in_specs=in_specs,
        out_specs=pl.BlockSpec(memory_space=pl.ANY),
        out_shape=jax.ShapeDtypeStruct((n_total, TEXT_HIDDEN), jnp.float32),
        input_output_aliases=aliases,
        scratch_shapes=[
            pltpu.VMEM((MM_HIDDEN, TEXT_HIDDEN), jnp.bfloat16),
            pltpu.VMEM((MM_HIDDEN, MM_HIDDEN), jnp.bfloat16),
            pltpu.VMEM((NUM_OUT_BUFS, ROW_BLOCK, TEXT_HIDDEN), jnp.float32),
            pltpu.SemaphoreType.DMA((NUM_OUT_BUFS,)),
        ],
        compiler_params=pltpu.CompilerParams(
            dimension_semantics=("arbitrary",),
        ),
    )(*args)


NUM_CHUNKS = 2


def kernel(input_ids, embedding_table, hard_norm_scale, projection_weight):
    b, s = input_ids.shape
    n = b * s
    ids_flat = input_ids.reshape(n)
    chunk = n // NUM_CHUNKS
    gathered = [
        _sc_gather(embedding_table, ids_flat[c * chunk:(c + 1) * chunk])
        for c in range(NUM_CHUNKS)
    ]
    out = None
    for c in range(NUM_CHUNKS):
        out = _tc_norm_proj_norm(
            gathered[c], hard_norm_scale, projection_weight,
            n, c * chunk, out,
        )
    return out.reshape(b, s, TEXT_HIDDEN)


# speedup vs baseline: 1.0342x; 1.0342x over previous
import functools

import jax
import jax.numpy as jnp
from jax.experimental import pallas as pl
from jax.experimental.pallas import tpu as pltpu
from jax.experimental.pallas import tpu_sc as plsc

MM_HIDDEN = 128
TEXT_HIDDEN = 2048
EPS = 1e-06

GATHER_WINDOW = 128
ROW_BLOCK = 1024
NUM_OUT_BUFS = 4


def _sc_gather(table, ids_flat):
    n = ids_flat.shape[0]
    ids2d = ids_flat.reshape(1, n)
    mesh = plsc.VectorSubcoreMesh(core_axis_name="core", subcore_axis_name="subcore")

    @pl.kernel(
        out_type=jax.ShapeDtypeStruct((n, MM_HIDDEN), table.dtype),
        mesh=mesh,
    )
    def gather_kernel(table_hbm, ids_hbm, out_hbm):
        def body(i_vmem, o_vmem):
            pltpu.sync_copy(table_hbm.at[i_vmem.at[0]], o_vmem)

        pltpu.emit_pipeline(
            body,
            grid=(n // GATHER_WINDOW,),
            in_specs=[pl.BlockSpec((1, GATHER_WINDOW), lambda i: (0, i))],
            out_specs=[pl.BlockSpec((GATHER_WINDOW, MM_HIDDEN), lambda i: (i, 0))],
            core_axis_name=("core", "subcore"),
            dimension_semantics=(pltpu.PARALLEL,),
        )(ids_hbm, out_hbm)

    return gather_kernel(table, ids2d)


def _tc_body(x_ref, s_ref, w_ref, o_hbm, w16_ref, g16_ref, bufs, sems,
             *, row_offset):
    @pl.when(pl.program_id(0) == 0)
    def _():
        w16 = w_ref[...].astype(jnp.bfloat16)
        w16_ref[...] = w16
        g = jax.lax.dot_general(
            w16, w16, (((1,), (1,)), ((), ())),
            preferred_element_type=jnp.float32,
        )
        g16_ref[...] = g.astype(jnp.bfloat16)

    x = x_ref[...]
    inv1 = jax.lax.rsqrt(jnp.mean(x * x, axis=1, keepdims=True) + EPS)
    y32 = x * inv1 * s_ref[...]
    y = y32.astype(jnp.bfloat16)
    t = jax.lax.dot_general(
        y, g16_ref[...], (((1,), (0,)), ((), ())),
        preferred_element_type=jnp.float32,
    )
    q = jnp.sum(t * y32, axis=1, keepdims=True)
    inv2 = jax.lax.rsqrt(q / TEXT_HIDDEN + EPS)
    y2 = (y32 * inv2).astype(jnp.bfloat16)
    z = jax.lax.dot_general(
        y2, w16_ref[...], (((1,), (0,)), ((), ())),
        preferred_element_type=jnp.float32,
    )

    i = pl.program_id(0)
    num_steps = pl.num_programs(0)
    slot = jax.lax.rem(i, NUM_OUT_BUFS)

    def _copy(s, step):
        return pltpu.make_async_copy(
            bufs.at[s],
            o_hbm.at[pl.ds(row_offset + step * ROW_BLOCK, ROW_BLOCK), :],
            sems.at[s],
        )

    @pl.when(i >= NUM_OUT_BUFS)
    def _():
        _copy(slot, i - NUM_OUT_BUFS).wait()

    bufs[slot] = z
    _copy(slot, i).start()

    @pl.when(i == num_steps - 1)
    def _():
        @pl.loop(0, NUM_OUT_BUFS)
        def _(j):
            step = num_steps - NUM_OUT_BUFS + j
            _copy(jax.lax.rem(step, NUM_OUT_BUFS), step).wait()


def _tc_norm_proj_norm(gathered, scale, weight, n_total, row_offset, out_prev):
    n = gathered.shape[0]
    body = functools.partial(_tc_body, row_offset=row_offset)
    in_specs = [
        pl.BlockSpec((ROW_BLOCK, MM_HIDDEN), lambda i: (i, 0)),
        pl.BlockSpec((1, MM_HIDDEN), lambda i: (0, 0)),
        pl.BlockSpec((MM_HIDDEN, TEXT_HIDDEN), lambda i: (0, 0)),
    ]
    args = [gathered, scale.reshape(1, MM_HIDDEN), weight]
    aliases = {}
    if out_prev is not None:
        in_specs.append(pl.BlockSpec(memory_space=pl.ANY))
        args.append(out_prev)
        aliases = {3: 0}

    def wrapped(*refs):
        if out_prev is not None:
            x_ref, s_ref, w_ref, _oin, o_hbm, *scratch = refs
        else:
            x_ref, s_ref, w_ref, o_hbm, *scratch = refs
        body(x_ref, s_ref, w_ref, o_hbm, *scratch)

    return pl.pallas_call(
        wrapped,
        grid=(n // ROW_BLOCK,),
        in_specs=in_specs,
        out_specs=pl.BlockSpec(memory_space=pl.ANY),
        out_shape=jax.ShapeDtypeStruct((n_total, TEXT_HIDDEN), jnp.float32),
        input_output_aliases=aliases,
        scratch_shapes=[
            pltpu.VMEM((MM_HIDDEN, TEXT_HIDDEN), jnp.bfloat16),
            pltpu.VMEM((MM_HIDDEN, MM_HIDDEN), jnp.bfloat16),
            pltpu.VMEM((NUM_OUT_BUFS, ROW_BLOCK, TEXT_HIDDEN), jnp.float32),
            pltpu.SemaphoreType.DMA((NUM_OUT_BUFS,)),
        ],
        compiler_params=pltpu.CompilerParams(
            dimension_semantics=("arbitrary",),
        ),
    )(*args)


NUM_CHUNKS = 1


def kernel(input_ids, embedding_table, hard_norm_scale, projection_weight):
    b, s = input_ids.shape
    n = b * s
    ids_flat = input_ids.reshape(n)
    chunk = n // NUM_CHUNKS
    gathered = [
        _sc_gather(embedding_table, ids_flat[c * chunk:(c + 1) * chunk])
        for c in range(NUM_CHUNKS)
    ]
    out = None
    for c in range(NUM_CHUNKS):
        out = _tc_norm_proj_norm(
            gathered[c], hard_norm_scale, projection_weight,
            n, c * chunk, out,
        )
    return out.reshape(b, s, TEXT_HIDDEN)


# speedup vs baseline: 1.0570x; 1.0221x over previous
import functools

import jax
import jax.numpy as jnp
from jax.experimental import pallas as pl
from jax.experimental.pallas import tpu as pltpu
from jax.experimental.pallas import tpu_sc as plsc

MM_HIDDEN = 128
TEXT_HIDDEN = 2048
EPS = 1e-06

GATHER_WINDOW = 128
ROW_BLOCK = 1024
NUM_OUT_BUFS = 4


SC_NUM_CORES = 2
SC_NUM_SUBCORES = 16


def _sc_gather(table, ids_flat):
    n = ids_flat.shape[0]
    nw = SC_NUM_CORES * SC_NUM_SUBCORES
    chunks_per_w = n // (nw * GATHER_WINDOW)
    ids2d = ids_flat.reshape(n // GATHER_WINDOW, GATHER_WINDOW)
    mesh = plsc.VectorSubcoreMesh(core_axis_name="c", subcore_axis_name="s")

    @functools.partial(
        pl.kernel,
        out_type=jax.ShapeDtypeStruct((n, MM_HIDDEN), table.dtype),
        mesh=mesh,
        scratch_types=[
            pltpu.VMEM((chunks_per_w, GATHER_WINDOW), jnp.int32),
            pltpu.VMEM((chunks_per_w, GATHER_WINDOW, MM_HIDDEN), table.dtype),
            pltpu.SemaphoreType.DMA((chunks_per_w,)),
            pltpu.SemaphoreType.DMA,
        ],
    )
    def gather_kernel(table_hbm, ids_hbm, out_hbm, idx_v, rows_v, gsems, wsem):
        wid = jax.lax.axis_index("s") * SC_NUM_CORES + jax.lax.axis_index("c")
        row0 = wid * chunks_per_w
        pltpu.sync_copy(ids_hbm.at[pl.ds(row0, chunks_per_w)], idx_v)
        gathers = [
            pltpu.make_async_copy(
                table_hbm.at[idx_v.at[c]], rows_v.at[c], gsems.at[c],
            )
            for c in range(chunks_per_w)
        ]
        writes = [
            pltpu.make_async_copy(
                rows_v.at[c],
                out_hbm.at[pl.ds((row0 + c) * GATHER_WINDOW, GATHER_WINDOW)],
                wsem,
            )
            for c in range(chunks_per_w)
        ]
        for g in gathers:
            g.start()
        for c in range(chunks_per_w):
            gathers[c].wait()
            writes[c].start()
        for w in writes:
            w.wait()

    return gather_kernel(table, ids2d)


def _tc_body(x_ref, s_ref, w_ref, o_hbm, w16_ref, g16_ref, bufs, sems,
             *, row_offset):
    @pl.when(pl.program_id(0) == 0)
    def _():
        w16 = w_ref[...].astype(jnp.bfloat16)
        w16_ref[...] = w16
        g = jax.lax.dot_general(
            w16, w16, (((1,), (1,)), ((), ())),
            preferred_element_type=jnp.float32,
        )
        g16_ref[...] = g.astype(jnp.bfloat16)

    x = x_ref[...]
    inv1 = jax.lax.rsqrt(jnp.mean(x * x, axis=1, keepdims=True) + EPS)
    y32 = x * inv1 * s_ref[...]
    y = y32.astype(jnp.bfloat16)
    t = jax.lax.dot_general(
        y, g16_ref[...], (((1,), (0,)), ((), ())),
        preferred_element_type=jnp.float32,
    )
    q = jnp.sum(t * y32, axis=1, keepdims=True)
    inv2 = jax.lax.rsqrt(q / TEXT_HIDDEN + EPS)
    y2 = (y32 * inv2).astype(jnp.bfloat16)
    z = jax.lax.dot_general(
        y2, w16_ref[...], (((1,), (0,)), ((), ())),
        preferred_element_type=jnp.float32,
    )

    i = pl.program_id(0)
    num_steps = pl.num_programs(0)
    slot = jax.lax.rem(i, NUM_OUT_BUFS)

    def _copy(s, step):
        return pltpu.make_async_copy(
            bufs.at[s],
            o_hbm.at[pl.ds(row_offset + step * ROW_BLOCK, ROW_BLOCK), :],
            sems.at[s],
        )

    @pl.when(i >= NUM_OUT_BUFS)
    def _():
        _copy(slot, i - NUM_OUT_BUFS).wait()

    bufs[slot] = z
    _copy(slot, i).start()

    @pl.when(i == num_steps - 1)
    def _():
        @pl.loop(0, NUM_OUT_BUFS)
        def _(j):
            step = num_steps - NUM_OUT_BUFS + j
            _copy(jax.lax.rem(step, NUM_OUT_BUFS), step).wait()


def _tc_norm_proj_norm(gathered, scale, weight, n_total, row_offset, out_prev):
    n = gathered.shape[0]
    body = functools.partial(_tc_body, row_offset=row_offset)
    in_specs = [
        pl.BlockSpec((ROW_BLOCK, MM_HIDDEN), lambda i: (i, 0)),
        pl.BlockSpec((1, MM_HIDDEN), lambda i: (0, 0)),
        pl.BlockSpec((MM_HIDDEN, TEXT_HIDDEN), lambda i: (0, 0)),
    ]
    args = [gathered, scale.reshape(1, MM_HIDDEN), weight]
    aliases = {}
    if out_prev is not None:
        in_specs.append(pl.BlockSpec(memory_space=pl.ANY))
        args.append(out_prev)
        aliases = {3: 0}

    def wrapped(*refs):
        if out_prev is not None:
            x_ref, s_ref, w_ref, _oin, o_hbm, *scratch = refs
        else:
            x_ref, s_ref, w_ref, o_hbm, *scratch = refs
        body(x_ref, s_ref, w_ref, o_hbm, *scratch)

    return pl.pallas_call(
        wrapped,
        grid=(n // ROW_BLOCK,),
        in_specs=in_specs,
        out_specs=pl.BlockSpec(memory_space=pl.ANY),
        out_shape=jax.ShapeDtypeStruct((n_total, TEXT_HIDDEN), jnp.float32),
        input_output_aliases=aliases,
        scratch_shapes=[
            pltpu.VMEM((MM_HIDDEN, TEXT_HIDDEN), jnp.bfloat16),
            pltpu.VMEM((MM_HIDDEN, MM_HIDDEN), jnp.bfloat16),
            pltpu.VMEM((NUM_OUT_BUFS, ROW_BLOCK, TEXT_HIDDEN), jnp.float32),
            pltpu.SemaphoreType.DMA((NUM_OUT_BUFS,)),
        ],
        compiler_params=pltpu.CompilerParams(
            dimension_semantics=("arbitrary",),
        ),
    )(*args)


NUM_CHUNKS = 1


def kernel(input_ids, embedding_table, hard_norm_scale, projection_weight):
    b, s = input_ids.shape
    n = b * s
    ids_flat = input_ids.reshape(n)
    chunk = n // NUM_CHUNKS
    gathered = [
        _sc_gather(embedding_table, ids_flat[c * chunk:(c + 1) * chunk])
        for c in range(NUM_CHUNKS)
    ]
    out = None
    for c in range(NUM_CHUNKS):
        out = _tc_norm_proj_norm(
            gathered[c], hard_norm_scale, projection_weight,
            n, c * chunk, out,
        )
    return out.reshape(b, s, TEXT_HIDDEN)
